# baseline (speedup 1.0000x reference)
import functools
import math

import jax
import jax.numpy as jnp
import numpy as np
from jax import lax
from jax.experimental import pallas as pl
from jax.experimental.pallas import tpu as pltpu
from jax.experimental.pallas import tpu_sc as plsc

TAU_FEAT = 0.2
TAU_SIM = 0.1
EPS = 1e-12
A_ATTN, A_ENT, A_MUT = 1.0, 0.4, 0.6
K_MAX = 64
TOK_TILE = 256
IDX_PAD = 128
RECIP_H = np.float32(0.083333336)


def _score_kernel(attn_ref, xt_ref, idx_ref):
    xt = xt_ref[0]
    C, Nt = xt.shape
    b = pl.program_id(0)

    t = xt / jnp.float32(TAU_FEAT)
    m = jnp.max(t, axis=0, keepdims=True)
    e = jnp.exp(t - m)
    p = e / jnp.sum(e, axis=0, keepdims=True)
    p = jnp.maximum(p, jnp.float32(EPS))
    hent = -jnp.sum(p * jnp.log(p), axis=0, keepdims=True) / math.log(C + EPS)

    nrm = jnp.sqrt(jnp.sum(xt * xt, axis=0, keepdims=True)) + jnp.float32(EPS)
    zt = xt / nrm

    ent_tiles = []
    for t0 in range(0, Nt, TOK_TILE):
        ztt = zt[:, t0:t0 + TOK_TILE]
        st = jax.lax.dot_general(zt, ztt, (((0,), (0,)), ((), ())),
                                 preferred_element_type=jnp.float32)
        rows = jax.lax.broadcasted_iota(jnp.int32, st.shape, 0)
        cols = jax.lax.broadcasted_iota(jnp.int32, st.shape, 1) + t0
        st = jnp.where(rows == cols, jnp.float32(-1e9), st)
        a = st / jnp.float32(TAU_SIM)
        m2 = jnp.max(a, axis=0, keepdims=True)
        e2 = jnp.exp(a - m2)
        q = e2 / jnp.sum(e2, axis=0, keepdims=True)
        q = jnp.maximum(q, jnp.float32(EPS))
        ent_tiles.append(-jnp.sum(q * jnp.log(q), axis=0, keepdims=True))
    hsim = jnp.concatenate(ent_tiles, axis=1) / math.log(Nt + EPS)
    mut = 1.0 - hsim

    v = attn_ref[0]
    tot = v[0:1]
    for h in range(1, v.shape[0]):
        tot = tot + v[h:h + 1]
    s_attn = tot * RECIP_H

    S = jnp.concatenate([s_attn.T, hent.T, mut.T], axis=1)
    means = jnp.sum(S, axis=0, keepdims=True) / jnp.float32(Nt)
    D = S - means
    std = jnp.sqrt(jnp.sum(D * D, axis=0, keepdims=True)
                   / jnp.float32(Nt - 1)) + jnp.float32(EPS)
    Zs = D / std
    fused = ((A_ATTN * Zs[:, 0:1] + A_ENT * Zs[:, 1:2])
             + A_MUT * Zs[:, 2:3]).T

    ii = jax.lax.broadcasted_iota(jnp.int32, (1, Nt), 1)
    slot = jax.lax.broadcasted_iota(jnp.int32, (1, IDX_PAD), 1)

    def body(k, carry):
        w, acc = carry
        mval = jnp.max(w)
        idx = jnp.min(jnp.where(w == mval, ii, Nt))
        acc = jnp.where(slot == k + 1, idx + 1, acc)
        return jnp.where(ii == idx, -jnp.inf, w), acc

    acc0 = jnp.zeros((1, IDX_PAD), jnp.int32)
    _, acc = jax.lax.fori_loop(0, K_MAX, body, (fused, acc0))
    idx_ref[...] = (acc + b * (Nt + 1))[None]


def _make_sc_gather(n_idx, C):
    info = plsc.get_sparse_core_info()
    NC, NS = info.num_cores, info.num_subcores
    NW = NC * NS
    assert n_idx % (8 * NW) == 0
    per_w = n_idx // NW
    mesh = plsc.VectorSubcoreMesh(core_axis_name="c", subcore_axis_name="s")

    @functools.partial(
        pl.kernel, mesh=mesh,
        out_type=jax.ShapeDtypeStruct((n_idx, C), jnp.float32),
        scratch_types=[
            pltpu.VMEM((per_w,), jnp.int32),
            pltpu.VMEM((per_w, C), jnp.float32),
            pltpu.SemaphoreType.DMA,
        ],
    )
    def gather_k(table_hbm, idx_hbm, out_hbm, idx_v, rows_v, sem):
        wid = lax.axis_index("s") * NC + lax.axis_index("c")
        base = wid * per_w
        pltpu.sync_copy(idx_hbm.at[pl.ds(base, per_w)], idx_v)
        pltpu.async_copy(table_hbm.at[idx_v], rows_v, sem).wait()
        pltpu.sync_copy(rows_v, out_hbm.at[pl.ds(base, per_w)])

    return gather_k


def kernel(hidden, attn, keys):
    B, N, C = hidden.shape
    H = attn.shape[1]
    Nt = N - 1
    attn_cls = attn[:, :, 0, 1:]
    xt = jnp.swapaxes(keys[:, 1:, :], 1, 2)

    idx = pl.pallas_call(
        _score_kernel,
        grid=(B,),
        in_specs=[
            pl.BlockSpec((1, H, Nt), lambda b: (b, 0, 0)),
            pl.BlockSpec((1, C, Nt), lambda b: (b, 0, 0)),
        ],
        out_specs=pl.BlockSpec((1, 1, IDX_PAD), lambda b: (b, 0, 0)),
        out_shape=jax.ShapeDtypeStruct((B, 1, IDX_PAD), jnp.int32),
    )(attn_cls, xt)

    C_PAD = 256
    table = jnp.pad(hidden.reshape(B * N, C), ((0, 0), (0, C_PAD - C)))
    flat_idx = idx.reshape(B * IDX_PAD)
    gathered = _make_sc_gather(B * IDX_PAD, C_PAD)(table, flat_idx)
    g = gathered.reshape(B, IDX_PAD, C_PAD)
    return g[:, :1 + K_MAX, :C]
